# Initial kernel scaffold; baseline (speedup 1.0000x reference)
import jax
import jax.numpy as jnp
from jax.experimental import pallas as pl


def kernel(X, emb_weights, free_term):
    raise NotImplementedError("write your pallas kernel here")


# speedup vs baseline: 96.2777x; 96.2777x over previous
import functools

import jax
import jax.numpy as jnp
from jax import lax
from jax.experimental import pallas as pl
from jax.experimental.pallas import tpu as pltpu
from jax.experimental.pallas import tpu_sc as plsc

_B = 16384
_F = 26
_V = 32
_NW = 32
_ROWS = _B // _NW
_CHUNK = _ROWS * _F
_L = 16
_JSTEPS = _ROWS // _L


@functools.cache
def _build_sc_kernel():
    mesh = plsc.VectorSubcoreMesh(core_axis_name="c", subcore_axis_name="s")

    @functools.partial(
        pl.kernel,
        mesh=mesh,
        out_type=jax.ShapeDtypeStruct((_B,), jnp.float32),
        compiler_params=pltpu.CompilerParams(needs_layout_passes=False),
        scratch_types=[
            pltpu.VMEM((_CHUNK,), jnp.int32),
            pltpu.VMEM((_F * _V,), jnp.float32),
            pltpu.VMEM((_ROWS,), jnp.float32),
        ],
    )
    def sc_embed_sum(x_hbm, tbl_hbm, out_hbm, x_v, tbl_v, out_v):
        wid = lax.axis_index("s") * 2 + lax.axis_index("c")
        pltpu.sync_copy(x_hbm.at[pl.ds(wid * _CHUNK, _CHUNK)], x_v)
        pltpu.sync_copy(tbl_hbm, tbl_v)
        lane_off = lax.iota(jnp.int32, 16) * _F

        def body(j, carry):
            base = j * (_L * _F) + lane_off
            acc = jnp.zeros((_L,), jnp.float32)
            for f in range(_F):
                xi = plsc.load_gather(x_v, [base + f])
                acc = acc + plsc.load_gather(tbl_v, [xi + f * _V])
            out_v[pl.ds(j * _L, _L)] = acc
            return carry

        lax.fori_loop(0, _JSTEPS, body, 0)
        pltpu.sync_copy(out_v, out_hbm.at[pl.ds(wid * _ROWS, _ROWS)])

    return sc_embed_sum


def kernel(X, emb_weights, free_term):
    tbl = emb_weights[:, :, 0].reshape(_F * _V)
    tbl = tbl.at[:_V].add(free_term[0])
    return _build_sc_kernel()(X.reshape(_B * _F), tbl)


# speedup vs baseline: 96.3030x; 1.0003x over previous
import functools

import jax
import jax.numpy as jnp
from jax import lax
from jax.experimental import pallas as pl
from jax.experimental.pallas import tpu as pltpu
from jax.experimental.pallas import tpu_sc as plsc

_B = 16384
_F = 26
_V = 32
_NW = 32
_ROWS = _B // _NW
_CHUNK = _ROWS * _F
_L = 16
_JSTEPS = _ROWS // _L


@functools.cache
def _build_sc_kernel():
    mesh = plsc.VectorSubcoreMesh(core_axis_name="c", subcore_axis_name="s")

    @functools.partial(
        pl.kernel,
        mesh=mesh,
        out_type=jax.ShapeDtypeStruct((_B,), jnp.float32),
        compiler_params=pltpu.CompilerParams(
            needs_layout_passes=False,
            disable_bounds_checks=True,
            disable_semaphore_checks=True,
            skip_device_barrier=True,
        ),
        scratch_types=[
            pltpu.VMEM((_CHUNK,), jnp.int32),
            pltpu.VMEM((_F * _V,), jnp.float32),
            pltpu.VMEM((_ROWS,), jnp.float32),
        ],
    )
    def sc_embed_sum(x_hbm, tbl_hbm, out_hbm, x_v, tbl_v, out_v):
        wid = lax.axis_index("s") * 2 + lax.axis_index("c")
        pltpu.sync_copy(x_hbm.at[pl.ds(wid * _CHUNK, _CHUNK)], x_v)
        pltpu.sync_copy(tbl_hbm, tbl_v)
        lane_off = lax.iota(jnp.int32, 16) * _F

        def body(j, carry):
            base = j * (_L * _F) + lane_off
            acc = jnp.zeros((_L,), jnp.float32)
            for f in range(_F):
                xi = plsc.load_gather(x_v, [base + f])
                acc = acc + plsc.load_gather(tbl_v, [xi + f * _V])
            out_v[pl.ds(j * _L, _L)] = acc
            return carry

        lax.fori_loop(0, _JSTEPS, body, 0)
        pltpu.sync_copy(out_v, out_hbm.at[pl.ds(wid * _ROWS, _ROWS)])

    return sc_embed_sum


def kernel(X, emb_weights, free_term):
    field0 = (jax.lax.iota(jnp.int32, _F) == 0).astype(jnp.float32)
    tbl = (emb_weights[:, :, 0] + free_term[0] * field0[:, None]).reshape(_F * _V)
    return _build_sc_kernel()(X.reshape(_B * _F), tbl)


# speedup vs baseline: 100.9708x; 1.0485x over previous
import functools

import jax
import jax.numpy as jnp
from jax import lax
from jax.experimental import pallas as pl
from jax.experimental.pallas import tpu as pltpu
from jax.experimental.pallas import tpu_sc as plsc

_B = 16384
_F = 26
_V = 32
_NW = 32
_ROWS = _B // _NW
_CHUNK = _ROWS * _F
_L = 16
_JSTEPS = _ROWS // _L


@functools.cache
def _build_sc_kernel():
    mesh = plsc.VectorSubcoreMesh(core_axis_name="c", subcore_axis_name="s")

    @functools.partial(
        pl.kernel,
        mesh=mesh,
        out_type=jax.ShapeDtypeStruct((_B,), jnp.float32),
        compiler_params=pltpu.CompilerParams(
            needs_layout_passes=False,
            disable_bounds_checks=True,
            disable_semaphore_checks=True,
            skip_device_barrier=True,
        ),
        scratch_types=[
            pltpu.VMEM((_ROWS, _F), jnp.int32),
            pltpu.VMEM((_F * _V,), jnp.float32),
            pltpu.VMEM((_ROWS,), jnp.float32),
        ],
    )
    def sc_embed_sum(x_hbm, tbl_hbm, out_hbm, x_v, tbl_v, out_v):
        wid = lax.axis_index("s") * 2 + lax.axis_index("c")
        pltpu.sync_copy(x_hbm.at[pl.ds(wid * _ROWS, _ROWS)], x_v)
        pltpu.sync_copy(tbl_hbm, tbl_v)
        lane = lax.iota(jnp.int32, 16)

        def body(j, carry):
            rvec = j * _L + lane
            acc = jnp.zeros((_L,), jnp.float32)
            for f in range(_F):
                fvec = jnp.full((_L,), f, jnp.int32)
                xi = plsc.load_gather(x_v, [rvec, fvec])
                acc = acc + plsc.load_gather(tbl_v, [xi + f * _V])
            out_v[pl.ds(j * _L, _L)] = acc
            return carry

        lax.fori_loop(0, _JSTEPS, body, 0)
        pltpu.sync_copy(out_v, out_hbm.at[pl.ds(wid * _ROWS, _ROWS)])

    return sc_embed_sum


def kernel(X, emb_weights, free_term):
    field0 = (jax.lax.iota(jnp.int32, _F) == 0).astype(jnp.float32)
    tbl = (emb_weights[:, :, 0] + free_term[0] * field0[:, None]).reshape(_F * _V)
    return _build_sc_kernel()(X, tbl)
